# baseline (speedup 1.0000x reference)
import functools
import jax
import jax.numpy as jnp
import numpy as np
from jax import lax
from jax.experimental import pallas as pl
from jax.experimental.pallas import tpu as pltpu
from jax.experimental.pallas import tpu_sc as plsc

N_NODES = 10000
N_EDGES = 320000
B = 1024
H = 128
P_IN = 1280
VOCAB = 1001
DATA_MIN = -11.355
DATA_MAX = 23.892

NC = 2
NS = 16
NW = NC * NS

NP = 10240
NPA = 10112
EP = 327680
EPW = EP // NW
CH = 128
NCHUNKS = EPW // CH
RPTA = NPA // NS

NODES_PER_W = NP // NW
NODE_CH = 80
NODE_CHUNKS = NODES_PER_W // NODE_CH

_BN_RSQRT = float(1.0 / np.sqrt(1.0 + 1e-5))


def _leaky(x):
    return jnp.where(x >= 0, x, 0.01 * x)


def _sc_mesh():
    return plsc.VectorSubcoreMesh(
        core_axis_name="c", subcore_axis_name="s", num_cores=NC, num_subcores=NS
    )


def _emb_body(emb_hbm, nodes_hbm, x0_hbm, idx_v, rbuf, sem):
    cid = lax.axis_index("c")
    sid = lax.axis_index("s")
    wid = cid * NS + sid
    for k in range(NODE_CHUNKS):
        base = pl.multiple_of(wid * NODES_PER_W + k * NODE_CH, 8)
        pltpu.sync_copy(nodes_hbm.at[pl.ds(base, NODE_CH)], idx_v)
        pltpu.async_copy(emb_hbm.at[idx_v], rbuf, sem).wait()
        pltpu.sync_copy(rbuf, x0_hbm.at[pl.ds(base, NODE_CH)])


def _emb_call(emb, nodes_p):
    fn = pl.kernel(
        _emb_body,
        out_type=jax.ShapeDtypeStruct((NP, H), jnp.float32),
        mesh=_sc_mesh(),
        scratch_types=[
            pltpu.VMEM((NODE_CH,), jnp.int32),
            pltpu.VMEM((NODE_CH, H), jnp.float32),
            pltpu.SemaphoreType.DMA,
        ],
    )
    return fn(emb, nodes_p)


NB = 2


NITER = NCHUNKS // NB


def _bf_to_f32(bfb, fbuf):
    def crow(r, _):
        for q in range(4):
            w = bfb[r, pl.ds(q * 16, 16)]
            lo = lax.bitcast_convert_type(w << 16, jnp.float32)
            hi = lax.bitcast_convert_type(w & jnp.int32(-65536), jnp.float32)
            fbuf[r, pl.ds(q * 16, 16)] = lo
            fbuf[r, pl.ds(q * 16 + 64, 16)] = hi
        return 0

    lax.fori_loop(0, CH, crow, 0)


def _spmm_body(xb_hbm, src_hbm, dst_hbm, agg_hbm,
               srcb, dstb, bfbufs, fbufs, agg_sh, gsems, isems, jsems, ssems):
    cid = lax.axis_index("c")
    sid = lax.axis_index("s")
    wid = cid * NS + sid
    e0 = wid * EPW

    def _idx_base(j, t):
        return pl.multiple_of(e0 + (j * NB + t) * CH, 8)

    def zrow(i, _):
        for j in range(H // 16):
            fbufs[0][i, pl.ds(j * 16, 16)] = jnp.zeros((16,), jnp.float32)
        return 0

    lax.fori_loop(0, CH, zrow, 0)
    r0 = sid * RPTA
    for k in range(4):
        pltpu.sync_copy(fbufs[0], agg_sh.at[pl.ds(r0 + k * CH, CH)])
    pltpu.sync_copy(
        fbufs[0].at[pl.ds(0, RPTA - 4 * CH)],
        agg_sh.at[pl.ds(r0 + 4 * CH, RPTA - 4 * CH)],
    )
    plsc.subcore_barrier()

    def _wait_scatter(t):
        pltpu.make_async_copy(agg_hbm.at[cid, pl.ds(0, CH)], fbufs[t], ssems[t]).wait()

    for t in range(NB):
        b = _idx_base(0, t)
        pltpu.async_copy(src_hbm.at[pl.ds(b, CH)], srcb[t], isems[t])
        pltpu.async_copy(dst_hbm.at[pl.ds(b, CH)], dstb[0][t], jsems[t])

    def do_iter(j, p):
        for t in range(NB):
            @pl.when(j > 0)
            def _():
                _wait_scatter(t)

            pltpu.make_async_copy(
                src_hbm.at[pl.ds(_idx_base(j, t), CH)], srcb[t], isems[t]
            ).wait()
            pltpu.async_copy(xb_hbm.at[srcb[t]], bfbufs[t], gsems[t])

            @pl.when(j < NITER - 1)
            def _():
                pltpu.async_copy(
                    dst_hbm.at[pl.ds(_idx_base(j + 1, t), CH)], dstb[1 - p][t],
                    jsems[t],
                )

        for t in range(NB):
            pltpu.make_async_copy(xb_hbm.at[srcb[t]], bfbufs[t], gsems[t]).wait()

            @pl.when(j < NITER - 1)
            def _():
                pltpu.async_copy(
                    src_hbm.at[pl.ds(_idx_base(j + 1, t), CH)], srcb[t], isems[t]
                )

            _bf_to_f32(bfbufs[t], fbufs[t])

            pltpu.make_async_copy(
                dst_hbm.at[pl.ds(_idx_base(j, t), CH)], dstb[p][t], jsems[t]
            ).wait()
            pltpu.async_copy(fbufs[t], agg_sh.at[dstb[p][t]], ssems[t], add=True)

    def ebody(jj, _):
        do_iter(jj * 2, 0)
        do_iter(jj * 2 + 1, 1)
        return 0

    lax.fori_loop(0, NITER // 2, ebody, 0)
    for t in range(NB):
        _wait_scatter(t)
    plsc.subcore_barrier()

    pltpu.sync_copy(
        agg_sh.at[pl.ds(r0, RPTA)],
        agg_hbm.at[cid, pl.ds(r0, RPTA)],
    )


def _spmm_call(xb, src, dst):
    fn = pl.kernel(
        _spmm_body,
        out_type=jax.ShapeDtypeStruct((NC, NP, H), jnp.float32),
        mesh=_sc_mesh(),
        scratch_types=[
            [pltpu.VMEM((CH,), jnp.int32) for _ in range(NB)],
            [[pltpu.VMEM((CH,), jnp.int32) for _ in range(NB)] for _ in range(2)],
            [pltpu.VMEM((CH, H // 2), jnp.int32) for _ in range(NB)],
            [pltpu.VMEM((CH, H), jnp.float32) for _ in range(NB)],
            pltpu.VMEM_SHARED((NPA, H), jnp.float32),
            [pltpu.SemaphoreType.DMA for _ in range(NB)],
            [pltpu.SemaphoreType.DMA for _ in range(NB)],
            [pltpu.SemaphoreType.DMA for _ in range(NB)],
            [pltpu.SemaphoreType.DMA for _ in range(NB)],
        ],
        compiler_params=pltpu.CompilerParams(use_tc_tiling_on_sc=False),
    )
    return fn(xb, src, dst)


def _perm_bf16(y):
    yb = y.astype(jnp.bfloat16)
    n = y.shape[0]
    return jnp.stack([yb[:, :64], yb[:, 64:]], axis=-1).reshape(n, H)


def _layer_body(agg_ref, x_ref, wr_ref, wt_ref, br_ref, o_ref, ob_ref):
    a = agg_ref[0] + agg_ref[1]
    y = (
        jnp.dot(a, wr_ref[...], preferred_element_type=jnp.float32)
        + jnp.dot(x_ref[...], wt_ref[...], preferred_element_type=jnp.float32)
        + br_ref[...]
    )
    o = _leaky(y)
    o_ref[...] = o
    ob_ref[...] = _perm_bf16(o)


def _layer_call(agg, x, WrT, WtT, br):
    grid = NP // 1024
    return pl.pallas_call(
        _layer_body,
        grid=(grid,),
        in_specs=[
            pl.BlockSpec((NC, 1024, H), lambda i: (0, i, 0)),
            pl.BlockSpec((1024, H), lambda i: (i, 0)),
            pl.BlockSpec((H, H), lambda i: (0, 0)),
            pl.BlockSpec((H, H), lambda i: (0, 0)),
            pl.BlockSpec((1, H), lambda i: (0, 0)),
        ],
        out_specs=[
            pl.BlockSpec((1024, H), lambda i: (i, 0)),
            pl.BlockSpec((1024, H), lambda i: (i, 0)),
        ],
        out_shape=[
            jax.ShapeDtypeStruct((NP, H), jnp.float32),
            jax.ShapeDtypeStruct((NP, H), jnp.bfloat16),
        ],
    )(agg, x, WrT, WtT, br)


def _layer3_body(agg_ref, x_ref, wr_ref, wt_ref, br_ref, batch_ref, bcol_ref,
                 sums_ref, cnt_ref):
    i = pl.program_id(0)

    @pl.when(i == 0)
    def _init():
        sums_ref[...] = jnp.zeros_like(sums_ref)
        cnt_ref[...] = jnp.zeros_like(cnt_ref)

    a = agg_ref[0] + agg_ref[1]
    y = (
        jnp.dot(a, wr_ref[...], preferred_element_type=jnp.float32)
        + jnp.dot(x_ref[...], wt_ref[...], preferred_element_type=jnp.float32)
        + br_ref[...]
    )
    x3 = _leaky(y)
    x3 = jnp.where(bcol_ref[0] < B, x3, 0.0)
    segs = lax.broadcasted_iota(jnp.int32, (B, 1), 0)
    sel = (batch_ref[0] == segs).astype(jnp.float32)
    sums_ref[...] += jnp.dot(sel, x3, preferred_element_type=jnp.float32)
    cnt_ref[...] += jnp.sum(sel, axis=1, keepdims=True)


def _layer3_call(agg, x, WrT, WtT, br, batch2d, batchcol):
    grid = NP // 1024
    return pl.pallas_call(
        _layer3_body,
        grid=(grid,),
        in_specs=[
            pl.BlockSpec((NC, 1024, H), lambda i: (0, i, 0)),
            pl.BlockSpec((1024, H), lambda i: (i, 0)),
            pl.BlockSpec((H, H), lambda i: (0, 0)),
            pl.BlockSpec((H, H), lambda i: (0, 0)),
            pl.BlockSpec((1, H), lambda i: (0, 0)),
            pl.BlockSpec((1, 1, 1024), lambda i: (i, 0, 0)),
            pl.BlockSpec((1, 1024, 1), lambda i: (i, 0, 0)),
        ],
        out_specs=[
            pl.BlockSpec((B, H), lambda i: (0, 0)),
            pl.BlockSpec((B, 1), lambda i: (0, 0)),
        ],
        out_shape=[
            jax.ShapeDtypeStruct((B, H), jnp.float32),
            jax.ShapeDtypeStruct((B, 1), jnp.float32),
        ],
    )(agg, x, WrT, WtT, br, batch2d, batchcol)


def _prot_body(prot_ref, w1t_ref, b1_ref, g1_ref, be1_ref,
               w2t_ref, b2_ref, g2_ref, be2_ref, h_ref):
    h = jnp.dot(prot_ref[...], w1t_ref[...], preferred_element_type=jnp.float32)
    h = _leaky(h + b1_ref[...])
    h = g1_ref[...] * h * _BN_RSQRT + be1_ref[...]
    h = jnp.dot(h, w2t_ref[...], preferred_element_type=jnp.float32)
    h = _leaky(h + b2_ref[...])
    h_ref[...] = g2_ref[...] * h * _BN_RSQRT + be2_ref[...]


def _prot_call(prot, W1T, b1, g1, be1, W2T, b2, g2, be2):
    blk = 256
    grid = B // blk
    return pl.pallas_call(
        _prot_body,
        grid=(grid,),
        in_specs=[
            pl.BlockSpec((blk, P_IN), lambda i: (i, 0)),
            pl.BlockSpec((P_IN, 400), lambda i: (0, 0)),
            pl.BlockSpec((1, 400), lambda i: (0, 0)),
            pl.BlockSpec((1, 400), lambda i: (0, 0)),
            pl.BlockSpec((1, 400), lambda i: (0, 0)),
            pl.BlockSpec((400, H), lambda i: (0, 0)),
            pl.BlockSpec((1, H), lambda i: (0, 0)),
            pl.BlockSpec((1, H), lambda i: (0, 0)),
            pl.BlockSpec((1, H), lambda i: (0, 0)),
        ],
        out_specs=pl.BlockSpec((blk, H), lambda i: (i, 0)),
        out_shape=jax.ShapeDtypeStruct((B, H), jnp.float32),
    )(prot, W1T, b1, g1, be1, W2T, b2, g2, be2)


def _final_body(
    h_ref, sums_ref, cnt_ref,
    w1h_ref, w1p_ref, bfc1_ref, gbn_ref, bebn_ref,
    wfc2_ref, bfc2_ref, o_ref,
):
    h = h_ref[...]
    pooled = sums_ref[...] / jnp.maximum(cnt_ref[...], 1.0)
    z = (
        jnp.dot(h, w1h_ref[...], preferred_element_type=jnp.float32)
        + jnp.dot(pooled, w1p_ref[...], preferred_element_type=jnp.float32)
        + bfc1_ref[...]
    )
    z = _leaky(gbn_ref[...] * z * _BN_RSQRT + bebn_ref[...])
    o = jnp.dot(z, wfc2_ref[...], preferred_element_type=jnp.float32) + bfc2_ref[...]
    o_ref[...] = jax.nn.sigmoid(o) * (DATA_MAX - DATA_MIN) + DATA_MIN


def _final_call(h, sums, cnt, W1h, W1p, bfc1, gbn, bebn, Wfc2T, bfc2):
    blk = 256
    grid = B // blk
    return pl.pallas_call(
        _final_body,
        grid=(grid,),
        in_specs=[
            pl.BlockSpec((blk, H), lambda i: (i, 0)),
            pl.BlockSpec((blk, H), lambda i: (i, 0)),
            pl.BlockSpec((blk, 1), lambda i: (i, 0)),
            pl.BlockSpec((H, 64), lambda i: (0, 0)),
            pl.BlockSpec((H, 64), lambda i: (0, 0)),
            pl.BlockSpec((1, 64), lambda i: (0, 0)),
            pl.BlockSpec((1, 64), lambda i: (0, 0)),
            pl.BlockSpec((1, 64), lambda i: (0, 0)),
            pl.BlockSpec((64, 1), lambda i: (0, 0)),
            pl.BlockSpec((1, 1), lambda i: (0, 0)),
        ],
        out_specs=pl.BlockSpec((blk, 1), lambda i: (i, 0)),
        out_shape=jax.ShapeDtypeStruct((B, 1), jnp.float32),
    )(h, sums, cnt, W1h, W1p, bfc1, gbn, bebn, Wfc2T, bfc2)


def kernel(prot, nodes, edge_index, batch, emb, W_pe1, b_pe1, W_pe2, b_pe2,
           g_bp1, be_bp1, g_bp2, be_bp2,
           Wrel1, brel1, Wroot1, Wrel2, brel2, Wroot2, Wrel3, brel3, Wroot3,
           W_fc1, b_fc1, g_bn1, be_bn1, W_fc2, b_fc2):
    e_pad = EP - N_EDGES
    ar = jnp.arange(e_pad, dtype=jnp.int32)
    src = jnp.concatenate([edge_index[0].astype(jnp.int32), ar % N_NODES])
    dst = jnp.concatenate([edge_index[1].astype(jnp.int32), N_NODES + ar % (NPA - N_NODES)])
    n_pad = NP - N_NODES
    nodes_p = jnp.concatenate(
        [nodes.astype(jnp.int32), jnp.arange(n_pad, dtype=jnp.int32) % VOCAB]
    )
    batch_p = jnp.concatenate([batch.astype(jnp.int32), jnp.full((n_pad,), B, jnp.int32)])
    batch2d = batch_p.reshape(NP // 1024, 1, 1024)
    batchcol = batch_p.reshape(NP // 1024, 1024, 1)

    def pack32(y):
        yb = jnp.stack([y[:, :H // 2], y[:, H // 2:]], axis=-1).astype(jnp.bfloat16)
        return jax.lax.bitcast_convert_type(yb, jnp.int32)

    row = lambda v: v.reshape(1, -1)

    h = _prot_call(
        prot,
        W_pe1.T, row(b_pe1), row(g_bp1), row(be_bp1),
        W_pe2.T, row(b_pe2), row(g_bp2), row(be_bp2),
    )

    def to32(yb):
        return jax.lax.bitcast_convert_type(yb.reshape(NP, H // 2, 2), jnp.int32)

    x0 = _emb_call(emb, nodes_p)

    a1 = _spmm_call(pack32(x0), src, dst)
    x1, x1b = _layer_call(a1, x0, Wrel1.T, Wroot1.T, row(brel1))
    a2 = _spmm_call(to32(x1b), src, dst)
    x2, x2b = _layer_call(a2, x1, Wrel2.T, Wroot2.T, row(brel2))
    a3 = _spmm_call(to32(x2b), src, dst)
    sums, cnt = _layer3_call(a3, x2, Wrel3.T, Wroot3.T, row(brel3), batch2d, batchcol)

    out = _final_call(
        h, sums, cnt,
        W_fc1[:, :H].T, W_fc1[:, H:].T, row(b_fc1), row(g_bn1), row(be_bn1),
        W_fc2.T, row(b_fc2),
    )
    return out


# speedup vs baseline: 2.2607x; 2.2607x over previous
import functools
import jax
import jax.numpy as jnp
import numpy as np
from jax import lax
from jax.experimental import pallas as pl
from jax.experimental.pallas import tpu as pltpu
from jax.experimental.pallas import tpu_sc as plsc

N_NODES = 10000
N_EDGES = 320000
B = 1024
H = 128
P_IN = 1280
VOCAB = 1001
DATA_MIN = -11.355
DATA_MAX = 23.892

NC = 2
NS = 16
NW = NC * NS

NP = 10240
EP = 327680
EPW = EP // NW
CH = 128
NCHUNKS = EPW // CH
ROWS_PER_TEC = NP // NS

NODES_PER_W = NP // NW
NODE_CH = 80
NODE_CHUNKS = NODES_PER_W // NODE_CH

_BN_RSQRT = float(1.0 / np.sqrt(1.0 + 1e-5))


def _leaky(x):
    return jnp.where(x >= 0, x, 0.01 * x)


def _sc_mesh():
    return plsc.VectorSubcoreMesh(
        core_axis_name="c", subcore_axis_name="s", num_cores=NC, num_subcores=NS
    )


def _emb_body(emb_hbm, nodes_hbm, x0_hbm, idx_v, rbuf, sem):
    cid = lax.axis_index("c")
    sid = lax.axis_index("s")
    wid = cid * NS + sid
    for k in range(NODE_CHUNKS):
        base = pl.multiple_of(wid * NODES_PER_W + k * NODE_CH, 8)
        pltpu.sync_copy(nodes_hbm.at[pl.ds(base, NODE_CH)], idx_v)
        pltpu.async_copy(emb_hbm.at[idx_v], rbuf, sem).wait()
        pltpu.sync_copy(rbuf, x0_hbm.at[pl.ds(base, NODE_CH)])


def _emb_call(emb, nodes_p):
    fn = pl.kernel(
        _emb_body,
        out_type=jax.ShapeDtypeStruct((NP, H), jnp.float32),
        mesh=_sc_mesh(),
        scratch_types=[
            pltpu.VMEM((NODE_CH,), jnp.int32),
            pltpu.VMEM((NODE_CH, H), jnp.float32),
            pltpu.SemaphoreType.DMA,
        ],
    )
    return fn(emb, nodes_p)


NB = 2


NITER = NCHUNKS // NB


def _spmm_body(x_hbm, src_hbm, dst_hbm, agg_hbm,
               srcb, dstb, gbufs, agg_sh, gsems, isems, jsems, ssems):
    cid = lax.axis_index("c")
    sid = lax.axis_index("s")
    wid = cid * NS + sid
    e0 = wid * EPW

    def _idx_base(j, t):
        return pl.multiple_of(e0 + (j * NB + t) * CH, 8)

    def zrow(i, _):
        for j in range(H // 16):
            gbufs[0][i, pl.ds(j * 16, 16)] = jnp.zeros((16,), jnp.float32)
        return 0

    lax.fori_loop(0, CH, zrow, 0)
    for k in range(ROWS_PER_TEC // CH):
        pltpu.sync_copy(gbufs[0], agg_sh.at[pl.ds(sid * ROWS_PER_TEC + k * CH, CH)])
    plsc.subcore_barrier()

    def _wait_scatter(t):
        pltpu.make_async_copy(x_hbm.at[pl.ds(0, CH)], gbufs[t], ssems[t]).wait()

    for t in range(NB):
        b = _idx_base(0, t)
        pltpu.async_copy(src_hbm.at[pl.ds(b, CH)], srcb[t], isems[t])
        pltpu.async_copy(dst_hbm.at[pl.ds(b, CH)], dstb[0][t], jsems[t])

    def do_iter(j, p):
        for t in range(NB):
            @pl.when(j > 0)
            def _():
                _wait_scatter(t)

            pltpu.make_async_copy(
                src_hbm.at[pl.ds(_idx_base(j, t), CH)], srcb[t], isems[t]
            ).wait()
            pltpu.async_copy(x_hbm.at[srcb[t]], gbufs[t], gsems[t])

            @pl.when(j < NITER - 1)
            def _():
                pltpu.async_copy(
                    dst_hbm.at[pl.ds(_idx_base(j + 1, t), CH)], dstb[1 - p][t],
                    jsems[t],
                )

        for t in range(NB):
            pltpu.make_async_copy(x_hbm.at[srcb[t]], gbufs[t], gsems[t]).wait()

            @pl.when(j < NITER - 1)
            def _():
                pltpu.async_copy(
                    src_hbm.at[pl.ds(_idx_base(j + 1, t), CH)], srcb[t], isems[t]
                )

            pltpu.make_async_copy(
                dst_hbm.at[pl.ds(_idx_base(j, t), CH)], dstb[p][t], jsems[t]
            ).wait()
            pltpu.async_copy(gbufs[t], agg_sh.at[dstb[p][t]], ssems[t], add=True)

    def ebody(jj, _):
        do_iter(jj * 2, 0)
        do_iter(jj * 2 + 1, 1)
        return 0

    lax.fori_loop(0, NITER // 2, ebody, 0)
    for t in range(NB):
        _wait_scatter(t)
    plsc.subcore_barrier()

    r0 = sid * ROWS_PER_TEC
    pltpu.sync_copy(
        agg_sh.at[pl.ds(r0, ROWS_PER_TEC)],
        agg_hbm.at[cid, pl.ds(r0, ROWS_PER_TEC)],
    )


def _spmm_call(x, src, dst):
    fn = pl.kernel(
        _spmm_body,
        out_type=jax.ShapeDtypeStruct((NC, NP, H), jnp.float32),
        mesh=_sc_mesh(),
        scratch_types=[
            [pltpu.VMEM((CH,), jnp.int32) for _ in range(NB)],
            [[pltpu.VMEM((CH,), jnp.int32) for _ in range(NB)] for _ in range(2)],
            [pltpu.VMEM((CH, H), jnp.float32) for _ in range(NB)],
            pltpu.VMEM_SHARED((NP, H), jnp.float32),
            [pltpu.SemaphoreType.DMA for _ in range(NB)],
            [pltpu.SemaphoreType.DMA for _ in range(NB)],
            [pltpu.SemaphoreType.DMA for _ in range(NB)],
            [pltpu.SemaphoreType.DMA for _ in range(NB)],
        ],
    )
    return fn(x, src, dst)


def _layer_body(agg_ref, x_ref, wr_ref, wt_ref, br_ref, o_ref):
    a = agg_ref[0] + agg_ref[1]
    y = (
        jnp.dot(a, wr_ref[...], preferred_element_type=jnp.float32)
        + jnp.dot(x_ref[...], wt_ref[...], preferred_element_type=jnp.float32)
        + br_ref[...]
    )
    o_ref[...] = _leaky(y)


def _layer_call(agg, x, WrT, WtT, br):
    grid = NP // 1024
    return pl.pallas_call(
        _layer_body,
        grid=(grid,),
        in_specs=[
            pl.BlockSpec((NC, 1024, H), lambda i: (0, i, 0)),
            pl.BlockSpec((1024, H), lambda i: (i, 0)),
            pl.BlockSpec((H, H), lambda i: (0, 0)),
            pl.BlockSpec((H, H), lambda i: (0, 0)),
            pl.BlockSpec((1, H), lambda i: (0, 0)),
        ],
        out_specs=pl.BlockSpec((1024, H), lambda i: (i, 0)),
        out_shape=jax.ShapeDtypeStruct((NP, H), jnp.float32),
    )(agg, x, WrT, WtT, br)


def _layer3_body(agg_ref, x_ref, wr_ref, wt_ref, br_ref, batch_ref, sums_ref, cnt_ref):
    i = pl.program_id(0)

    @pl.when(i == 0)
    def _init():
        sums_ref[...] = jnp.zeros_like(sums_ref)
        cnt_ref[...] = jnp.zeros_like(cnt_ref)

    a = agg_ref[0] + agg_ref[1]
    y = (
        jnp.dot(a, wr_ref[...], preferred_element_type=jnp.float32)
        + jnp.dot(x_ref[...], wt_ref[...], preferred_element_type=jnp.float32)
        + br_ref[...]
    )
    x3 = _leaky(y)
    segs = lax.broadcasted_iota(jnp.int32, (B, 1), 0)
    sel = (batch_ref[0] == segs).astype(jnp.float32)
    sums_ref[...] += jnp.dot(sel, x3, preferred_element_type=jnp.float32)
    cnt_ref[...] += jnp.sum(sel, axis=1, keepdims=True)


def _layer3_call(agg, x, WrT, WtT, br, batch2d):
    grid = NP // 1024
    return pl.pallas_call(
        _layer3_body,
        grid=(grid,),
        in_specs=[
            pl.BlockSpec((NC, 1024, H), lambda i: (0, i, 0)),
            pl.BlockSpec((1024, H), lambda i: (i, 0)),
            pl.BlockSpec((H, H), lambda i: (0, 0)),
            pl.BlockSpec((H, H), lambda i: (0, 0)),
            pl.BlockSpec((1, H), lambda i: (0, 0)),
            pl.BlockSpec((1, 1, 1024), lambda i: (i, 0, 0)),
        ],
        out_specs=[
            pl.BlockSpec((B, H), lambda i: (0, 0)),
            pl.BlockSpec((B, 1), lambda i: (0, 0)),
        ],
        out_shape=[
            jax.ShapeDtypeStruct((B, H), jnp.float32),
            jax.ShapeDtypeStruct((B, 1), jnp.float32),
        ],
    )(agg, x, WrT, WtT, br, batch2d)


def _prot_body(prot_ref, w1t_ref, b1_ref, g1_ref, be1_ref,
               w2t_ref, b2_ref, g2_ref, be2_ref, h_ref):
    h = jnp.dot(prot_ref[...], w1t_ref[...], preferred_element_type=jnp.float32)
    h = _leaky(h + b1_ref[...])
    h = g1_ref[...] * h * _BN_RSQRT + be1_ref[...]
    h = jnp.dot(h, w2t_ref[...], preferred_element_type=jnp.float32)
    h = _leaky(h + b2_ref[...])
    h_ref[...] = g2_ref[...] * h * _BN_RSQRT + be2_ref[...]


def _prot_call(prot, W1T, b1, g1, be1, W2T, b2, g2, be2):
    blk = 256
    grid = B // blk
    return pl.pallas_call(
        _prot_body,
        grid=(grid,),
        in_specs=[
            pl.BlockSpec((blk, P_IN), lambda i: (i, 0)),
            pl.BlockSpec((P_IN, 400), lambda i: (0, 0)),
            pl.BlockSpec((1, 400), lambda i: (0, 0)),
            pl.BlockSpec((1, 400), lambda i: (0, 0)),
            pl.BlockSpec((1, 400), lambda i: (0, 0)),
            pl.BlockSpec((400, H), lambda i: (0, 0)),
            pl.BlockSpec((1, H), lambda i: (0, 0)),
            pl.BlockSpec((1, H), lambda i: (0, 0)),
            pl.BlockSpec((1, H), lambda i: (0, 0)),
        ],
        out_specs=pl.BlockSpec((blk, H), lambda i: (i, 0)),
        out_shape=jax.ShapeDtypeStruct((B, H), jnp.float32),
    )(prot, W1T, b1, g1, be1, W2T, b2, g2, be2)


def _final_body(
    h_ref, sums_ref, cnt_ref,
    w1h_ref, w1p_ref, bfc1_ref, gbn_ref, bebn_ref,
    wfc2_ref, bfc2_ref, o_ref,
):
    h = h_ref[...]
    pooled = sums_ref[...] / jnp.maximum(cnt_ref[...], 1.0)
    z = (
        jnp.dot(h, w1h_ref[...], preferred_element_type=jnp.float32)
        + jnp.dot(pooled, w1p_ref[...], preferred_element_type=jnp.float32)
        + bfc1_ref[...]
    )
    z = _leaky(gbn_ref[...] * z * _BN_RSQRT + bebn_ref[...])
    o = jnp.dot(z, wfc2_ref[...], preferred_element_type=jnp.float32) + bfc2_ref[...]
    o_ref[...] = jax.nn.sigmoid(o) * (DATA_MAX - DATA_MIN) + DATA_MIN


def _final_call(h, sums, cnt, W1h, W1p, bfc1, gbn, bebn, Wfc2T, bfc2):
    blk = 256
    grid = B // blk
    return pl.pallas_call(
        _final_body,
        grid=(grid,),
        in_specs=[
            pl.BlockSpec((blk, H), lambda i: (i, 0)),
            pl.BlockSpec((blk, H), lambda i: (i, 0)),
            pl.BlockSpec((blk, 1), lambda i: (i, 0)),
            pl.BlockSpec((H, 64), lambda i: (0, 0)),
            pl.BlockSpec((H, 64), lambda i: (0, 0)),
            pl.BlockSpec((1, 64), lambda i: (0, 0)),
            pl.BlockSpec((1, 64), lambda i: (0, 0)),
            pl.BlockSpec((1, 64), lambda i: (0, 0)),
            pl.BlockSpec((64, 1), lambda i: (0, 0)),
            pl.BlockSpec((1, 1), lambda i: (0, 0)),
        ],
        out_specs=pl.BlockSpec((blk, 1), lambda i: (i, 0)),
        out_shape=jax.ShapeDtypeStruct((B, 1), jnp.float32),
    )(h, sums, cnt, W1h, W1p, bfc1, gbn, bebn, Wfc2T, bfc2)


def kernel(prot, nodes, edge_index, batch, emb, W_pe1, b_pe1, W_pe2, b_pe2,
           g_bp1, be_bp1, g_bp2, be_bp2,
           Wrel1, brel1, Wroot1, Wrel2, brel2, Wroot2, Wrel3, brel3, Wroot3,
           W_fc1, b_fc1, g_bn1, be_bn1, W_fc2, b_fc2):
    e_pad = EP - N_EDGES
    ar = jnp.arange(e_pad, dtype=jnp.int32)
    src = jnp.concatenate([edge_index[0].astype(jnp.int32), ar % N_NODES])
    dst = jnp.concatenate([edge_index[1].astype(jnp.int32), N_NODES + ar % (NP - N_NODES)])
    n_pad = NP - N_NODES
    nodes_p = jnp.concatenate(
        [nodes.astype(jnp.int32), jnp.arange(n_pad, dtype=jnp.int32) % VOCAB]
    )
    batch_p = jnp.concatenate([batch.astype(jnp.int32), jnp.full((n_pad,), B, jnp.int32)])
    batch2d = batch_p.reshape(NP // 1024, 1, 1024)

    row = lambda v: v.reshape(1, -1)

    h = _prot_call(
        prot,
        W_pe1.T, row(b_pe1), row(g_bp1), row(be_bp1),
        W_pe2.T, row(b_pe2), row(g_bp2), row(be_bp2),
    )

    x0 = _emb_call(emb, nodes_p)

    a1 = _spmm_call(x0, src, dst)
    x1 = _layer_call(a1, x0, Wrel1.T, Wroot1.T, row(brel1))
    a2 = _spmm_call(x1, src, dst)
    x2 = _layer_call(a2, x1, Wrel2.T, Wroot2.T, row(brel2))
    a3 = _spmm_call(x2, src, dst)
    sums, cnt = _layer3_call(a3, x2, Wrel3.T, Wroot3.T, row(brel3), batch2d)

    out = _final_call(
        h, sums, cnt,
        W_fc1[:, :H].T, W_fc1[:, H:].T, row(b_fc1), row(g_bn1), row(be_bn1),
        W_fc2.T, row(b_fc2),
    )
    return out


# speedup vs baseline: 2.7414x; 1.2126x over previous
import functools
import jax
import jax.numpy as jnp
import numpy as np
from jax import lax
from jax.experimental import pallas as pl
from jax.experimental.pallas import tpu as pltpu
from jax.experimental.pallas import tpu_sc as plsc

N_NODES = 10000
N_EDGES = 320000
B = 1024
H = 128
P_IN = 1280
VOCAB = 1001
DATA_MIN = -11.355
DATA_MAX = 23.892

NC = 2
NS = 16
NW = NC * NS

NP = 10240
EP = 327680
EPW = EP // NW
CH = 64
NCHUNKS = EPW // CH
ROWS_PER_TEC = NP // NS

NODES_PER_W = NP // NW
NODE_CH = 80
NODE_CHUNKS = NODES_PER_W // NODE_CH

_BN_RSQRT = float(1.0 / np.sqrt(1.0 + 1e-5))


def _leaky(x):
    return jnp.where(x >= 0, x, 0.01 * x)


def _sc_mesh():
    return plsc.VectorSubcoreMesh(
        core_axis_name="c", subcore_axis_name="s", num_cores=NC, num_subcores=NS
    )


def _emb_body(emb_hbm, nodes_hbm, x0_hbm, idx_v, rbuf, sem):
    cid = lax.axis_index("c")
    sid = lax.axis_index("s")
    wid = cid * NS + sid
    for k in range(NODE_CHUNKS):
        base = pl.multiple_of(wid * NODES_PER_W + k * NODE_CH, 8)
        pltpu.sync_copy(nodes_hbm.at[pl.ds(base, NODE_CH)], idx_v)
        pltpu.async_copy(emb_hbm.at[idx_v], rbuf, sem).wait()
        pltpu.sync_copy(rbuf, x0_hbm.at[pl.ds(base, NODE_CH)])


def _emb_call(emb, nodes_p):
    fn = pl.kernel(
        _emb_body,
        out_type=jax.ShapeDtypeStruct((NP, H), jnp.float32),
        mesh=_sc_mesh(),
        scratch_types=[
            pltpu.VMEM((NODE_CH,), jnp.int32),
            pltpu.VMEM((NODE_CH, H), jnp.float32),
            pltpu.SemaphoreType.DMA,
        ],
    )
    return fn(emb, nodes_p)


NB = 4


NITER = NCHUNKS // NB


def _spmm_body(x_hbm, src_hbm, dst_hbm, agg_hbm,
               srcb, dstb, gbufs, agg_sh, gsems, isems, jsems, ssems):
    cid = lax.axis_index("c")
    sid = lax.axis_index("s")
    wid = cid * NS + sid
    e0 = wid * EPW

    def _idx_base(j, t):
        return pl.multiple_of(e0 + (j * NB + t) * CH, 8)

    def zrow(i, _):
        for j in range(H // 16):
            gbufs[0][i, pl.ds(j * 16, 16)] = jnp.zeros((16,), jnp.float32)
        return 0

    lax.fori_loop(0, CH, zrow, 0)
    for k in range(ROWS_PER_TEC // CH):
        pltpu.sync_copy(gbufs[0], agg_sh.at[pl.ds(sid * ROWS_PER_TEC + k * CH, CH)])
    plsc.subcore_barrier()

    def _wait_scatter(t):
        pltpu.make_async_copy(x_hbm.at[pl.ds(0, CH)], gbufs[t], ssems[t]).wait()

    for t in range(NB):
        b = _idx_base(0, t)
        pltpu.async_copy(src_hbm.at[pl.ds(b, CH)], srcb[t], isems[t])
        pltpu.async_copy(dst_hbm.at[pl.ds(b, CH)], dstb[0][t], jsems[t])

    def do_iter(j, p):
        for t in range(NB):
            @pl.when(j > 0)
            def _():
                _wait_scatter(t)

            pltpu.make_async_copy(
                src_hbm.at[pl.ds(_idx_base(j, t), CH)], srcb[t], isems[t]
            ).wait()
            pltpu.async_copy(x_hbm.at[srcb[t]], gbufs[t], gsems[t])

            @pl.when(j < NITER - 1)
            def _():
                pltpu.async_copy(
                    dst_hbm.at[pl.ds(_idx_base(j + 1, t), CH)], dstb[1 - p][t],
                    jsems[t],
                )

        for t in range(NB):
            pltpu.make_async_copy(x_hbm.at[srcb[t]], gbufs[t], gsems[t]).wait()

            @pl.when(j < NITER - 1)
            def _():
                pltpu.async_copy(
                    src_hbm.at[pl.ds(_idx_base(j + 1, t), CH)], srcb[t], isems[t]
                )

            pltpu.make_async_copy(
                dst_hbm.at[pl.ds(_idx_base(j, t), CH)], dstb[p][t], jsems[t]
            ).wait()
            pltpu.async_copy(gbufs[t], agg_sh.at[dstb[p][t]], ssems[t], add=True)

    def ebody(jj, _):
        do_iter(jj * 2, 0)
        do_iter(jj * 2 + 1, 1)
        return 0

    lax.fori_loop(0, NITER // 2, ebody, 0)
    for t in range(NB):
        _wait_scatter(t)
    plsc.subcore_barrier()

    r0 = sid * ROWS_PER_TEC
    pltpu.sync_copy(
        agg_sh.at[pl.ds(r0, ROWS_PER_TEC)],
        agg_hbm.at[cid, pl.ds(r0, ROWS_PER_TEC)],
    )


def _spmm_call(x, src, dst):
    fn = pl.kernel(
        _spmm_body,
        out_type=jax.ShapeDtypeStruct((NC, NP, H), jnp.float32),
        mesh=_sc_mesh(),
        scratch_types=[
            [pltpu.VMEM((CH,), jnp.int32) for _ in range(NB)],
            [[pltpu.VMEM((CH,), jnp.int32) for _ in range(NB)] for _ in range(2)],
            [pltpu.VMEM((CH, H), jnp.float32) for _ in range(NB)],
            pltpu.VMEM_SHARED((NP, H), jnp.float32),
            [pltpu.SemaphoreType.DMA for _ in range(NB)],
            [pltpu.SemaphoreType.DMA for _ in range(NB)],
            [pltpu.SemaphoreType.DMA for _ in range(NB)],
            [pltpu.SemaphoreType.DMA for _ in range(NB)],
        ],
    )
    return fn(x, src, dst)


def _layer_body(agg_ref, x_ref, wr_ref, wt_ref, br_ref, o_ref):
    a = agg_ref[0] + agg_ref[1]
    y = (
        jnp.dot(a, wr_ref[...], preferred_element_type=jnp.float32)
        + jnp.dot(x_ref[...], wt_ref[...], preferred_element_type=jnp.float32)
        + br_ref[...]
    )
    o_ref[...] = _leaky(y)


def _layer_call(agg, x, WrT, WtT, br):
    grid = NP // 1024
    return pl.pallas_call(
        _layer_body,
        grid=(grid,),
        in_specs=[
            pl.BlockSpec((NC, 1024, H), lambda i: (0, i, 0)),
            pl.BlockSpec((1024, H), lambda i: (i, 0)),
            pl.BlockSpec((H, H), lambda i: (0, 0)),
            pl.BlockSpec((H, H), lambda i: (0, 0)),
            pl.BlockSpec((1, H), lambda i: (0, 0)),
        ],
        out_specs=pl.BlockSpec((1024, H), lambda i: (i, 0)),
        out_shape=jax.ShapeDtypeStruct((NP, H), jnp.float32),
    )(agg, x, WrT, WtT, br)


def _layer3_body(agg_ref, x_ref, wr_ref, wt_ref, br_ref, batch_ref, sums_ref, cnt_ref):
    i = pl.program_id(0)

    @pl.when(i == 0)
    def _init():
        sums_ref[...] = jnp.zeros_like(sums_ref)
        cnt_ref[...] = jnp.zeros_like(cnt_ref)

    a = agg_ref[0] + agg_ref[1]
    y = (
        jnp.dot(a, wr_ref[...], preferred_element_type=jnp.float32)
        + jnp.dot(x_ref[...], wt_ref[...], preferred_element_type=jnp.float32)
        + br_ref[...]
    )
    x3 = _leaky(y)
    segs = lax.broadcasted_iota(jnp.int32, (B, 1), 0)
    sel = (batch_ref[0] == segs).astype(jnp.float32)
    sums_ref[...] += jnp.dot(sel, x3, preferred_element_type=jnp.float32)
    cnt_ref[...] += jnp.sum(sel, axis=1, keepdims=True)


def _layer3_call(agg, x, WrT, WtT, br, batch2d):
    grid = NP // 1024
    return pl.pallas_call(
        _layer3_body,
        grid=(grid,),
        in_specs=[
            pl.BlockSpec((NC, 1024, H), lambda i: (0, i, 0)),
            pl.BlockSpec((1024, H), lambda i: (i, 0)),
            pl.BlockSpec((H, H), lambda i: (0, 0)),
            pl.BlockSpec((H, H), lambda i: (0, 0)),
            pl.BlockSpec((1, H), lambda i: (0, 0)),
            pl.BlockSpec((1, 1, 1024), lambda i: (i, 0, 0)),
        ],
        out_specs=[
            pl.BlockSpec((B, H), lambda i: (0, 0)),
            pl.BlockSpec((B, 1), lambda i: (0, 0)),
        ],
        out_shape=[
            jax.ShapeDtypeStruct((B, H), jnp.float32),
            jax.ShapeDtypeStruct((B, 1), jnp.float32),
        ],
    )(agg, x, WrT, WtT, br, batch2d)


def _prot_body(prot_ref, w1t_ref, b1_ref, g1_ref, be1_ref,
               w2t_ref, b2_ref, g2_ref, be2_ref, h_ref):
    h = jnp.dot(prot_ref[...], w1t_ref[...], preferred_element_type=jnp.float32)
    h = _leaky(h + b1_ref[...])
    h = g1_ref[...] * h * _BN_RSQRT + be1_ref[...]
    h = jnp.dot(h, w2t_ref[...], preferred_element_type=jnp.float32)
    h = _leaky(h + b2_ref[...])
    h_ref[...] = g2_ref[...] * h * _BN_RSQRT + be2_ref[...]


def _prot_call(prot, W1T, b1, g1, be1, W2T, b2, g2, be2):
    blk = 256
    grid = B // blk
    return pl.pallas_call(
        _prot_body,
        grid=(grid,),
        in_specs=[
            pl.BlockSpec((blk, P_IN), lambda i: (i, 0)),
            pl.BlockSpec((P_IN, 400), lambda i: (0, 0)),
            pl.BlockSpec((1, 400), lambda i: (0, 0)),
            pl.BlockSpec((1, 400), lambda i: (0, 0)),
            pl.BlockSpec((1, 400), lambda i: (0, 0)),
            pl.BlockSpec((400, H), lambda i: (0, 0)),
            pl.BlockSpec((1, H), lambda i: (0, 0)),
            pl.BlockSpec((1, H), lambda i: (0, 0)),
            pl.BlockSpec((1, H), lambda i: (0, 0)),
        ],
        out_specs=pl.BlockSpec((blk, H), lambda i: (i, 0)),
        out_shape=jax.ShapeDtypeStruct((B, H), jnp.float32),
    )(prot, W1T, b1, g1, be1, W2T, b2, g2, be2)


def _final_body(
    h_ref, sums_ref, cnt_ref,
    w1h_ref, w1p_ref, bfc1_ref, gbn_ref, bebn_ref,
    wfc2_ref, bfc2_ref, o_ref,
):
    h = h_ref[...]
    pooled = sums_ref[...] / jnp.maximum(cnt_ref[...], 1.0)
    z = (
        jnp.dot(h, w1h_ref[...], preferred_element_type=jnp.float32)
        + jnp.dot(pooled, w1p_ref[...], preferred_element_type=jnp.float32)
        + bfc1_ref[...]
    )
    z = _leaky(gbn_ref[...] * z * _BN_RSQRT + bebn_ref[...])
    o = jnp.dot(z, wfc2_ref[...], preferred_element_type=jnp.float32) + bfc2_ref[...]
    o_ref[...] = jax.nn.sigmoid(o) * (DATA_MAX - DATA_MIN) + DATA_MIN


def _final_call(h, sums, cnt, W1h, W1p, bfc1, gbn, bebn, Wfc2T, bfc2):
    blk = 256
    grid = B // blk
    return pl.pallas_call(
        _final_body,
        grid=(grid,),
        in_specs=[
            pl.BlockSpec((blk, H), lambda i: (i, 0)),
            pl.BlockSpec((blk, H), lambda i: (i, 0)),
            pl.BlockSpec((blk, 1), lambda i: (i, 0)),
            pl.BlockSpec((H, 64), lambda i: (0, 0)),
            pl.BlockSpec((H, 64), lambda i: (0, 0)),
            pl.BlockSpec((1, 64), lambda i: (0, 0)),
            pl.BlockSpec((1, 64), lambda i: (0, 0)),
            pl.BlockSpec((1, 64), lambda i: (0, 0)),
            pl.BlockSpec((64, 1), lambda i: (0, 0)),
            pl.BlockSpec((1, 1), lambda i: (0, 0)),
        ],
        out_specs=pl.BlockSpec((blk, 1), lambda i: (i, 0)),
        out_shape=jax.ShapeDtypeStruct((B, 1), jnp.float32),
    )(h, sums, cnt, W1h, W1p, bfc1, gbn, bebn, Wfc2T, bfc2)


def kernel(prot, nodes, edge_index, batch, emb, W_pe1, b_pe1, W_pe2, b_pe2,
           g_bp1, be_bp1, g_bp2, be_bp2,
           Wrel1, brel1, Wroot1, Wrel2, brel2, Wroot2, Wrel3, brel3, Wroot3,
           W_fc1, b_fc1, g_bn1, be_bn1, W_fc2, b_fc2):
    e_pad = EP - N_EDGES
    ar = jnp.arange(e_pad, dtype=jnp.int32)
    src = jnp.concatenate([edge_index[0].astype(jnp.int32), ar % N_NODES])
    dst = jnp.concatenate([edge_index[1].astype(jnp.int32), N_NODES + ar % (NP - N_NODES)])
    n_pad = NP - N_NODES
    nodes_p = jnp.concatenate(
        [nodes.astype(jnp.int32), jnp.arange(n_pad, dtype=jnp.int32) % VOCAB]
    )
    batch_p = jnp.concatenate([batch.astype(jnp.int32), jnp.full((n_pad,), B, jnp.int32)])
    batch2d = batch_p.reshape(NP // 1024, 1, 1024)

    row = lambda v: v.reshape(1, -1)

    h = _prot_call(
        prot,
        W_pe1.T, row(b_pe1), row(g_bp1), row(be_bp1),
        W_pe2.T, row(b_pe2), row(g_bp2), row(be_bp2),
    )

    x0 = _emb_call(emb, nodes_p)

    a1 = _spmm_call(x0, src, dst)
    x1 = _layer_call(a1, x0, Wrel1.T, Wroot1.T, row(brel1))
    a2 = _spmm_call(x1, src, dst)
    x2 = _layer_call(a2, x1, Wrel2.T, Wroot2.T, row(brel2))
    a3 = _spmm_call(x2, src, dst)
    sums, cnt = _layer3_call(a3, x2, Wrel3.T, Wroot3.T, row(brel3), batch2d)

    out = _final_call(
        h, sums, cnt,
        W_fc1[:, :H].T, W_fc1[:, H:].T, row(b_fc1), row(g_bn1), row(be_bn1),
        W_fc2.T, row(b_fc2),
    )
    return out
